# baseline (device time: 823607 ns/iter reference)
import jax
import jax.numpy as jnp
from jax import lax
from jax.experimental import pallas as pl
from jax.experimental.pallas import tpu as pltpu

N_DEV = 32
M_PER = 64
D = 1024
H_PER = 2048
N_STRIPS = 4
STRIP = (N_DEV * M_PER) // N_STRIPS


def kernel(x, Win0, Wout0, Win1, Wout1, Win2, Wout2):
    xb = x.astype(jnp.bfloat16)
    wins = [w.astype(jnp.bfloat16) for w in (Win0, Win1, Win2)]
    wouts = [w.astype(jnp.bfloat16) for w in (Wout0, Wout1, Wout2)]

    def body(x_ref, win0_ref, win1_ref, win2_ref,
             wout0_ref, wout1_ref, wout2_ref, out_ref,
             xfull_ref, p_ref, sendbuf_ref, rsrecv_ref,
             send_sem, ag_sems, rs_sems):
        me = lax.axis_index("i")
        left = lax.rem(me + (N_DEV - 1), N_DEV)
        right = lax.rem(me + 1, N_DEV)

        barrier_sem = pltpu.get_barrier_semaphore()
        for nbr in (left, right):
            pl.semaphore_signal(
                barrier_sem, inc=1,
                device_id=(nbr,), device_id_type=pl.DeviceIdType.MESH,
            )
        pl.semaphore_wait(barrier_sem, 2)

        win_refs = (win0_ref, win1_ref, win2_ref)
        wout_refs = (wout0_ref, wout1_ref, wout2_ref)

        xfull_ref[pl.ds(me * M_PER, M_PER), :] = x_ref[:, :]

        for l in range(3):
            def ag_step(h, carry):
                bj = lax.rem(me - h + N_DEV, N_DEV)
                rdma = pltpu.make_async_remote_copy(
                    src_ref=xfull_ref.at[pl.ds(bj * M_PER, M_PER)],
                    dst_ref=xfull_ref.at[pl.ds(bj * M_PER, M_PER)],
                    send_sem=send_sem,
                    recv_sem=ag_sems.at[h],
                    device_id=(right,),
                    device_id_type=pl.DeviceIdType.MESH,
                )
                rdma.start()
                rdma.wait()
                return carry

            lax.fori_loop(0, N_DEV - 1, ag_step, 0)

            for s in range(N_STRIPS):
                xs = xfull_ref[pl.ds(s * STRIP, STRIP), :]
                h = jnp.dot(xs, win_refs[l][:, :],
                            preferred_element_type=jnp.float32)
                h = jnp.maximum(h, 0.0).astype(jnp.bfloat16)
                p_ref[pl.ds(s * STRIP, STRIP), :] = jnp.dot(
                    h, wout_refs[l][:, :], preferred_element_type=jnp.float32)

            def rs_step(s, carry):
                bi = lax.rem(me - s + (N_DEV - 1), N_DEV)
                own = p_ref[pl.ds(bi * M_PER, M_PER), :]
                sprev = jnp.maximum(s - 1, 0)
                prev = rsrecv_ref[pl.ds(sprev * M_PER, M_PER), :]
                sendbuf_ref[:, :] = jnp.where(s == 0, own, own + prev)
                rdma = pltpu.make_async_remote_copy(
                    src_ref=sendbuf_ref,
                    dst_ref=rsrecv_ref.at[pl.ds(s * M_PER, M_PER)],
                    send_sem=send_sem,
                    recv_sem=rs_sems.at[s],
                    device_id=(right,),
                    device_id_type=pl.DeviceIdType.MESH,
                )
                rdma.start()
                rdma.wait()
                return carry

            lax.fori_loop(0, N_DEV - 1, rs_step, 0)

            xnew = (rsrecv_ref[pl.ds((N_DEV - 2) * M_PER, M_PER), :]
                    + p_ref[pl.ds(me * M_PER, M_PER), :])
            if l < 2:
                xfull_ref[pl.ds(me * M_PER, M_PER), :] = xnew.astype(
                    jnp.bfloat16)
            else:
                out_ref[:, :] = xnew

    return pl.pallas_call(
        body,
        out_shape=jax.ShapeDtypeStruct((M_PER, D), jnp.float32),
        in_specs=[pl.BlockSpec(memory_space=pltpu.VMEM)] * 7,
        out_specs=pl.BlockSpec(memory_space=pltpu.VMEM),
        scratch_shapes=[
            pltpu.VMEM((N_DEV * M_PER, D), jnp.bfloat16),
            pltpu.VMEM((N_DEV * M_PER, D), jnp.float32),
            pltpu.VMEM((M_PER, D), jnp.float32),
            pltpu.VMEM(((N_DEV - 1) * M_PER, D), jnp.float32),
            pltpu.SemaphoreType.DMA,
            pltpu.SemaphoreType.DMA((N_DEV - 1,)),
            pltpu.SemaphoreType.DMA((N_DEV - 1,)),
        ],
        compiler_params=pltpu.CompilerParams(collective_id=0),
    )(xb, wins[0], wins[1], wins[2], wouts[0], wouts[1], wouts[2])


# device time: 665054 ns/iter; 1.2384x vs baseline; 1.2384x over previous
import jax
import jax.numpy as jnp
from jax import lax
from jax.experimental import pallas as pl
from jax.experimental.pallas import tpu as pltpu

N_DEV = 32
M_PER = 64
D = 1024
N_STRIPS = 4
STRIP = (N_DEV * M_PER) // N_STRIPS

R_HOPS = N_DEV // 2
L_HOPS = N_DEV // 2 - 1


def kernel(x, Win0, Wout0, Win1, Wout1, Win2, Wout2):
    xb = x.astype(jnp.bfloat16)
    wins = [w.astype(jnp.bfloat16) for w in (Win0, Win1, Win2)]
    wouts = [w.astype(jnp.bfloat16) for w in (Wout0, Wout1, Wout2)]

    def body(x_ref, win0_ref, win1_ref, win2_ref,
             wout0_ref, wout1_ref, wout2_ref, out_ref,
             xfull_ref, p_ref, sbuf_r_ref, sbuf_l_ref,
             rsr_ref, rsl_ref,
             send_sem_r, send_sem_l, ag_r_sems, ag_l_sems,
             rs_r_sems, rs_l_sems):
        me = lax.axis_index("i")
        left = lax.rem(me + (N_DEV - 1), N_DEV)
        right = lax.rem(me + 1, N_DEV)

        barrier_sem = pltpu.get_barrier_semaphore()
        for nbr in (left, right):
            pl.semaphore_signal(
                barrier_sem, inc=1,
                device_id=(nbr,), device_id_type=pl.DeviceIdType.MESH,
            )
        pl.semaphore_wait(barrier_sem, 2)

        win_refs = (win0_ref, win1_ref, win2_ref)
        wout_refs = (wout0_ref, wout1_ref, wout2_ref)

        def ag_rdma_right(h):
            bj = lax.rem(me - h + N_DEV, N_DEV)
            return pltpu.make_async_remote_copy(
                src_ref=xfull_ref.at[pl.ds(bj * M_PER, M_PER)],
                dst_ref=xfull_ref.at[pl.ds(bj * M_PER, M_PER)],
                send_sem=send_sem_r,
                recv_sem=ag_r_sems.at[h],
                device_id=(right,),
                device_id_type=pl.DeviceIdType.MESH,
            )

        def ag_rdma_left(h):
            bj = lax.rem(me + h, N_DEV)
            return pltpu.make_async_remote_copy(
                src_ref=xfull_ref.at[pl.ds(bj * M_PER, M_PER)],
                dst_ref=xfull_ref.at[pl.ds(bj * M_PER, M_PER)],
                send_sem=send_sem_l,
                recv_sem=ag_l_sems.at[h],
                device_id=(left,),
                device_id_type=pl.DeviceIdType.MESH,
            )

        def rs_send_right(s):
            bi = lax.rem(me + L_HOPS - s, N_DEV)
            own = p_ref[pl.ds(bi * M_PER, M_PER), :]
            sprev = jnp.maximum(s - 1, 0)
            prev = rsr_ref[pl.ds(sprev * M_PER, M_PER), :]
            sbuf_r_ref[:, :] = jnp.where(s == 0, own, own + prev)
            return pltpu.make_async_remote_copy(
                src_ref=sbuf_r_ref,
                dst_ref=rsr_ref.at[pl.ds(s * M_PER, M_PER)],
                send_sem=send_sem_r,
                recv_sem=rs_r_sems.at[s],
                device_id=(right,),
                device_id_type=pl.DeviceIdType.MESH,
            )

        def rs_send_left(s):
            bi = lax.rem(me - R_HOPS + s + N_DEV, N_DEV)
            own = p_ref[pl.ds(bi * M_PER, M_PER), :]
            sprev = jnp.maximum(s - 1, 0)
            prev = rsl_ref[pl.ds(sprev * M_PER, M_PER), :]
            sbuf_l_ref[:, :] = jnp.where(s == 0, own, own + prev)
            return pltpu.make_async_remote_copy(
                src_ref=sbuf_l_ref,
                dst_ref=rsl_ref.at[pl.ds(s * M_PER, M_PER)],
                send_sem=send_sem_l,
                recv_sem=rs_l_sems.at[s],
                device_id=(left,),
                device_id_type=pl.DeviceIdType.MESH,
            )

        xfull_ref[pl.ds(me * M_PER, M_PER), :] = x_ref[:, :]

        for l in range(3):
            def ag_step(h, carry):
                r = ag_rdma_right(h)
                lft = ag_rdma_left(h)
                r.start()
                lft.start()
                r.wait()
                lft.wait()
                return carry

            lax.fori_loop(0, L_HOPS, ag_step, 0)
            r_tail = ag_rdma_right(R_HOPS - 1)
            r_tail.start()
            r_tail.wait()

            for s in range(N_STRIPS):
                xs = xfull_ref[pl.ds(s * STRIP, STRIP), :]
                h = jnp.dot(xs, win_refs[l][:, :],
                            preferred_element_type=jnp.float32)
                h = jnp.maximum(h, 0.0).astype(jnp.bfloat16)
                p_ref[pl.ds(s * STRIP, STRIP), :] = jnp.dot(
                    h, wout_refs[l][:, :], preferred_element_type=jnp.float32)

            def rs_step(s, carry):
                r = rs_send_right(s)
                lft = rs_send_left(s)
                r.start()
                lft.start()
                r.wait()
                lft.wait()
                return carry

            lax.fori_loop(0, L_HOPS, rs_step, 0)
            l_tail = rs_send_left(R_HOPS - 1)
            l_tail.start()
            l_tail.wait()

            xnew = (p_ref[pl.ds(me * M_PER, M_PER), :]
                    + rsr_ref[pl.ds((L_HOPS - 1) * M_PER, M_PER), :]
                    + rsl_ref[pl.ds((R_HOPS - 1) * M_PER, M_PER), :])
            if l < 2:
                xfull_ref[pl.ds(me * M_PER, M_PER), :] = xnew.astype(
                    jnp.bfloat16)
            else:
                out_ref[:, :] = xnew

    return pl.pallas_call(
        body,
        out_shape=jax.ShapeDtypeStruct((M_PER, D), jnp.float32),
        in_specs=[pl.BlockSpec(memory_space=pltpu.VMEM)] * 7,
        out_specs=pl.BlockSpec(memory_space=pltpu.VMEM),
        scratch_shapes=[
            pltpu.VMEM((N_DEV * M_PER, D), jnp.bfloat16),
            pltpu.VMEM((N_DEV * M_PER, D), jnp.float32),
            pltpu.VMEM((M_PER, D), jnp.float32),
            pltpu.VMEM((M_PER, D), jnp.float32),
            pltpu.VMEM((L_HOPS * M_PER, D), jnp.float32),
            pltpu.VMEM((R_HOPS * M_PER, D), jnp.float32),
            pltpu.SemaphoreType.DMA,
            pltpu.SemaphoreType.DMA,
            pltpu.SemaphoreType.DMA((R_HOPS,)),
            pltpu.SemaphoreType.DMA((L_HOPS,)),
            pltpu.SemaphoreType.DMA((L_HOPS,)),
            pltpu.SemaphoreType.DMA((R_HOPS,)),
        ],
        compiler_params=pltpu.CompilerParams(collective_id=0),
    )(xb, wins[0], wins[1], wins[2], wouts[0], wouts[1], wouts[2])


# device time: 509827 ns/iter; 1.6155x vs baseline; 1.3045x over previous
import jax
import jax.numpy as jnp
from jax import lax
from jax.experimental import pallas as pl
from jax.experimental.pallas import tpu as pltpu

N_DEV = 32
M_PER = 64
HALF = M_PER // 2
D = 1024

R_HOPS = N_DEV // 2
L_HOPS = N_DEV // 2 - 1


def kernel(x, Win0, Wout0, Win1, Wout1, Win2, Wout2):
    xb = x.astype(jnp.bfloat16)
    wins = [w.astype(jnp.bfloat16) for w in (Win0, Win1, Win2)]
    wouts = [w.astype(jnp.bfloat16) for w in (Wout0, Wout1, Wout2)]

    def body(x_ref, win0_ref, win1_ref, win2_ref,
             wout0_ref, wout1_ref, wout2_ref, out_ref,
             xfull_ref, p_ref, sbuf_r_ref, sbuf_l_ref,
             rsr_ref, rsl_ref,
             ag_sr_sems, ag_sl_sems, rs_sr_sems, rs_sl_sems,
             ag_r_sems, ag_l_sems, rs_r_sems, rs_l_sems):
        me = lax.axis_index("i")
        left = lax.rem(me + (N_DEV - 1), N_DEV)
        right = lax.rem(me + 1, N_DEV)

        barrier_sem = pltpu.get_barrier_semaphore()
        for nbr in (left, right):
            pl.semaphore_signal(
                barrier_sem, inc=1,
                device_id=(nbr,), device_id_type=pl.DeviceIdType.MESH,
            )
        pl.semaphore_wait(barrier_sem, 2)

        win_refs = (win0_ref, win1_ref, win2_ref)
        wout_refs = (wout0_ref, wout1_ref, wout2_ref)

        def ag_rdma(direction, half, h):
            if direction == 0:
                bj = lax.rem(me - h + N_DEV, N_DEV)
                tgt, ssem, rsem = right, ag_sr_sems, ag_r_sems
                n_steps = R_HOPS
            else:
                bj = lax.rem(me + h, N_DEV)
                tgt, ssem, rsem = left, ag_sl_sems, ag_l_sems
                n_steps = L_HOPS
            rows = pl.ds(bj * M_PER + half * HALF, HALF)
            return pltpu.make_async_remote_copy(
                src_ref=xfull_ref.at[rows],
                dst_ref=xfull_ref.at[rows],
                send_sem=ssem.at[half],
                recv_sem=rsem.at[half * n_steps + h],
                device_id=(tgt,),
                device_id_type=pl.DeviceIdType.MESH,
            )

        def rs_rdma(direction, half, s):
            if direction == 0:
                tgt, ssem, rsem = right, rs_sr_sems, rs_r_sems
                sbuf, slots, n_steps = sbuf_r_ref, rsr_ref, L_HOPS
            else:
                tgt, ssem, rsem = left, rs_sl_sems, rs_l_sems
                sbuf, slots, n_steps = sbuf_l_ref, rsl_ref, R_HOPS
            return pltpu.make_async_remote_copy(
                src_ref=sbuf.at[pl.ds(half * HALF, HALF)],
                dst_ref=slots.at[pl.ds(s * M_PER + half * HALF, HALF)],
                send_sem=ssem.at[half],
                recv_sem=rsem.at[half * n_steps + s],
                device_id=(tgt,),
                device_id_type=pl.DeviceIdType.MESH,
            )

        def compute_block(b, l):
            rows = pl.ds(b * M_PER, M_PER)
            xs = xfull_ref[rows, :]
            hh = jnp.dot(xs, win_refs[l][:, :],
                         preferred_element_type=jnp.float32)
            hh = jnp.maximum(hh, 0.0).astype(jnp.bfloat16)
            p_ref[rows, :] = jnp.dot(hh, wout_refs[l][:, :],
                                     preferred_element_type=jnp.float32)

        xfull_ref[pl.ds(me * M_PER, M_PER), :] = x_ref[:, :]

        for l in range(3):
            def ag_step(h, carry):
                for half in range(2):
                    @pl.when(h > 0)
                    def _():
                        ag_rdma(0, half, h - 1).wait()
                    ag_rdma(0, half, h).start()
                    @pl.when(h > 0)
                    def _():
                        ag_rdma(1, half, h - 1).wait()

                    @pl.when(h < L_HOPS)
                    def _():
                        ag_rdma(1, half, h).start()

                @pl.when(h == 0)
                def _():
                    compute_block(me, l)

                @pl.when(h > 0)
                def _():
                    compute_block(lax.rem(me - h + N_DEV, N_DEV), l)
                    compute_block(lax.rem(me + h, N_DEV), l)

                return carry

            lax.fori_loop(0, R_HOPS, ag_step, 0, unroll=False)
            for half in range(2):
                ag_rdma(0, half, R_HOPS - 1).wait()
            compute_block(lax.rem(me - R_HOPS + N_DEV, N_DEV), l)

            def rs_val(direction, half, s):
                if direction == 0:
                    bi = lax.rem(me + L_HOPS - s, N_DEV)
                    slots = rsr_ref
                else:
                    bi = lax.rem(me - R_HOPS + s + N_DEV, N_DEV)
                    slots = rsl_ref
                rows = pl.ds(bi * M_PER + half * HALF, HALF)
                own = p_ref[rows, :]
                sprev = jnp.maximum(s - 1, 0)
                prev = slots[pl.ds(sprev * M_PER + half * HALF, HALF), :]
                return jnp.where(s == 0, own, own + prev)

            def rs_step(s, carry):
                for half in range(2):
                    @pl.when(s > 0)
                    def _():
                        rs_rdma(1, half, s - 1).wait()
                    sbuf_l_ref[pl.ds(half * HALF, HALF), :] = rs_val(1, half, s)
                    rs_rdma(1, half, s).start()
                    @pl.when(s > 0)
                    def _():
                        rs_rdma(0, half, s - 1).wait()

                    @pl.when(s < L_HOPS)
                    def _():
                        sbuf_r_ref[pl.ds(half * HALF, HALF), :] = rs_val(
                            0, half, s)
                        rs_rdma(0, half, s).start()

                return carry

            lax.fori_loop(0, R_HOPS, rs_step, 0, unroll=False)
            for half in range(2):
                rs_rdma(1, half, R_HOPS - 1).wait()

            xnew = (p_ref[pl.ds(me * M_PER, M_PER), :]
                    + rsr_ref[pl.ds((L_HOPS - 1) * M_PER, M_PER), :]
                    + rsl_ref[pl.ds((R_HOPS - 1) * M_PER, M_PER), :])
            if l < 2:
                xfull_ref[pl.ds(me * M_PER, M_PER), :] = xnew.astype(
                    jnp.bfloat16)
            else:
                out_ref[:, :] = xnew

    return pl.pallas_call(
        body,
        out_shape=jax.ShapeDtypeStruct((M_PER, D), jnp.float32),
        in_specs=[pl.BlockSpec(memory_space=pltpu.VMEM)] * 7,
        out_specs=pl.BlockSpec(memory_space=pltpu.VMEM),
        scratch_shapes=[
            pltpu.VMEM((N_DEV * M_PER, D), jnp.bfloat16),
            pltpu.VMEM((N_DEV * M_PER, D), jnp.float32),
            pltpu.VMEM((M_PER, D), jnp.float32),
            pltpu.VMEM((M_PER, D), jnp.float32),
            pltpu.VMEM((L_HOPS * M_PER, D), jnp.float32),
            pltpu.VMEM((R_HOPS * M_PER, D), jnp.float32),
            pltpu.SemaphoreType.DMA((2,)),
            pltpu.SemaphoreType.DMA((2,)),
            pltpu.SemaphoreType.DMA((2,)),
            pltpu.SemaphoreType.DMA((2,)),
            pltpu.SemaphoreType.DMA((2 * R_HOPS,)),
            pltpu.SemaphoreType.DMA((2 * L_HOPS,)),
            pltpu.SemaphoreType.DMA((2 * L_HOPS,)),
            pltpu.SemaphoreType.DMA((2 * R_HOPS,)),
        ],
        compiler_params=pltpu.CompilerParams(collective_id=0),
    )(xb, wins[0], wins[1], wins[2], wouts[0], wouts[1], wouts[2])


# device time: 340012 ns/iter; 2.4223x vs baseline; 1.4994x over previous
import jax
import jax.numpy as jnp
from jax import lax
from jax.experimental import pallas as pl
from jax.experimental.pallas import tpu as pltpu

N_DEV = 32
M_PER = 64
HALF = M_PER // 2
D = 1024

R_HOPS = N_DEV // 2
L_HOPS = N_DEV // 2 - 1

CYCLE = [0, 3, 4, 7, 15, 12, 11, 8, 16, 19, 20, 23, 31, 28, 27, 24,
         25, 26, 29, 30, 22, 21, 18, 17, 9, 10, 13, 14, 6, 5, 2, 1]


def kernel(x, Win0, Wout0, Win1, Wout1, Win2, Wout2):
    xb = x.astype(jnp.bfloat16)
    wins = [w.astype(jnp.bfloat16) for w in (Win0, Win1, Win2)]
    wouts = [w.astype(jnp.bfloat16) for w in (Wout0, Wout1, Wout2)]
    cyc_tbl = jnp.asarray(CYCLE, dtype=jnp.int32).reshape(1, N_DEV)
    inv = [0] * N_DEV
    for k, v in enumerate(CYCLE):
        inv[v] = k
    inv_tbl = jnp.asarray(inv, dtype=jnp.int32).reshape(1, N_DEV)

    def body(cyc_ref, inv_ref, x_ref, win0_ref, win1_ref, win2_ref,
             wout0_ref, wout1_ref, wout2_ref, out_ref,
             xfull_ref, p_ref, sbuf_r_ref, sbuf_l_ref,
             rsr_ref, rsl_ref,
             ag_sr_sems, ag_sl_sems, rs_sr_sems, rs_sl_sems,
             ag_r_sems, ag_l_sems, rs_r_sems, rs_l_sems):
        me = lax.axis_index("i")

        def cyc(k):
            return cyc_ref[0, lax.rem(k + 2 * N_DEV, N_DEV)]

        c_me = inv_ref[0, me]
        left = cyc(c_me - 1)
        right = cyc(c_me + 1)

        barrier_sem = pltpu.get_barrier_semaphore()
        for nbr in (left, right):
            pl.semaphore_signal(
                barrier_sem, inc=1,
                device_id=(nbr,), device_id_type=pl.DeviceIdType.MESH,
            )
        pl.semaphore_wait(barrier_sem, 2)

        win_refs = (win0_ref, win1_ref, win2_ref)
        wout_refs = (wout0_ref, wout1_ref, wout2_ref)

        def ag_rdma(direction, half, h):
            if direction == 0:
                bj = cyc(c_me - h)
                tgt, ssem, rsem = right, ag_sr_sems, ag_r_sems
                n_steps = R_HOPS
            else:
                bj = cyc(c_me + h)
                tgt, ssem, rsem = left, ag_sl_sems, ag_l_sems
                n_steps = L_HOPS
            rows = pl.ds(bj * M_PER + half * HALF, HALF)
            return pltpu.make_async_remote_copy(
                src_ref=xfull_ref.at[rows],
                dst_ref=xfull_ref.at[rows],
                send_sem=ssem.at[half],
                recv_sem=rsem.at[half * n_steps + h],
                device_id=(tgt,),
                device_id_type=pl.DeviceIdType.MESH,
            )

        def rs_rdma(direction, half, s):
            if direction == 0:
                tgt, ssem, rsem = right, rs_sr_sems, rs_r_sems
                sbuf, slots, n_steps = sbuf_r_ref, rsr_ref, L_HOPS
            else:
                tgt, ssem, rsem = left, rs_sl_sems, rs_l_sems
                sbuf, slots, n_steps = sbuf_l_ref, rsl_ref, R_HOPS
            return pltpu.make_async_remote_copy(
                src_ref=sbuf.at[pl.ds(half * HALF, HALF)],
                dst_ref=slots.at[pl.ds(s * M_PER + half * HALF, HALF)],
                send_sem=ssem.at[half],
                recv_sem=rsem.at[half * n_steps + s],
                device_id=(tgt,),
                device_id_type=pl.DeviceIdType.MESH,
            )

        def compute_block(b, l):
            rows = pl.ds(b * M_PER, M_PER)
            xs = xfull_ref[rows, :]
            hh = jnp.dot(xs, win_refs[l][:, :],
                         preferred_element_type=jnp.float32)
            hh = jnp.maximum(hh, 0.0).astype(jnp.bfloat16)
            p_ref[rows, :] = jnp.dot(hh, wout_refs[l][:, :],
                                     preferred_element_type=jnp.float32)

        xfull_ref[pl.ds(me * M_PER, M_PER), :] = x_ref[:, :]

        for l in range(3):
            def ag_step(h, carry):
                for half in range(2):
                    @pl.when(h > 0)
                    def _():
                        ag_rdma(0, half, h - 1).wait()
                    ag_rdma(0, half, h).start()
                    @pl.when(h > 0)
                    def _():
                        ag_rdma(1, half, h - 1).wait()

                    @pl.when(h < L_HOPS)
                    def _():
                        ag_rdma(1, half, h).start()

                @pl.when(h == 0)
                def _():
                    compute_block(me, l)

                @pl.when(h > 0)
                def _():
                    compute_block(cyc(c_me - h), l)
                    compute_block(cyc(c_me + h), l)

                return carry

            lax.fori_loop(0, R_HOPS, ag_step, 0, unroll=False)
            for half in range(2):
                ag_rdma(0, half, R_HOPS - 1).wait()
            compute_block(cyc(c_me - R_HOPS), l)

            def rs_val(direction, half, s):
                if direction == 0:
                    bi = cyc(c_me + L_HOPS - s)
                    slots = rsr_ref
                else:
                    bi = cyc(c_me - R_HOPS + s)
                    slots = rsl_ref
                rows = pl.ds(bi * M_PER + half * HALF, HALF)
                own = p_ref[rows, :]
                sprev = jnp.maximum(s - 1, 0)
                prev = slots[pl.ds(sprev * M_PER + half * HALF, HALF), :]
                return jnp.where(s == 0, own, own + prev)

            def rs_step(s, carry):
                for half in range(2):
                    @pl.when(s > 0)
                    def _():
                        rs_rdma(1, half, s - 1).wait()
                    sbuf_l_ref[pl.ds(half * HALF, HALF), :] = rs_val(1, half, s)
                    rs_rdma(1, half, s).start()
                    @pl.when(s > 0)
                    def _():
                        rs_rdma(0, half, s - 1).wait()

                    @pl.when(s < L_HOPS)
                    def _():
                        sbuf_r_ref[pl.ds(half * HALF, HALF), :] = rs_val(
                            0, half, s)
                        rs_rdma(0, half, s).start()

                return carry

            lax.fori_loop(0, R_HOPS, rs_step, 0, unroll=False)
            for half in range(2):
                rs_rdma(1, half, R_HOPS - 1).wait()

            xnew = (p_ref[pl.ds(me * M_PER, M_PER), :]
                    + rsr_ref[pl.ds((L_HOPS - 1) * M_PER, M_PER), :]
                    + rsl_ref[pl.ds((R_HOPS - 1) * M_PER, M_PER), :])
            if l < 2:
                xfull_ref[pl.ds(me * M_PER, M_PER), :] = xnew.astype(
                    jnp.bfloat16)
            else:
                out_ref[:, :] = xnew

    return pl.pallas_call(
        body,
        out_shape=jax.ShapeDtypeStruct((M_PER, D), jnp.float32),
        in_specs=[pl.BlockSpec(memory_space=pltpu.SMEM)] * 2
        + [pl.BlockSpec(memory_space=pltpu.VMEM)] * 7,
        out_specs=pl.BlockSpec(memory_space=pltpu.VMEM),
        scratch_shapes=[
            pltpu.VMEM((N_DEV * M_PER, D), jnp.bfloat16),
            pltpu.VMEM((N_DEV * M_PER, D), jnp.float32),
            pltpu.VMEM((M_PER, D), jnp.float32),
            pltpu.VMEM((M_PER, D), jnp.float32),
            pltpu.VMEM((L_HOPS * M_PER, D), jnp.float32),
            pltpu.VMEM((R_HOPS * M_PER, D), jnp.float32),
            pltpu.SemaphoreType.DMA((2,)),
            pltpu.SemaphoreType.DMA((2,)),
            pltpu.SemaphoreType.DMA((2,)),
            pltpu.SemaphoreType.DMA((2,)),
            pltpu.SemaphoreType.DMA((2 * R_HOPS,)),
            pltpu.SemaphoreType.DMA((2 * L_HOPS,)),
            pltpu.SemaphoreType.DMA((2 * L_HOPS,)),
            pltpu.SemaphoreType.DMA((2 * R_HOPS,)),
        ],
        compiler_params=pltpu.CompilerParams(collective_id=0),
    )(cyc_tbl, inv_tbl, xb, wins[0], wins[1], wins[2],
      wouts[0], wouts[1], wouts[2])


# device time: 304375 ns/iter; 2.7059x vs baseline; 1.1171x over previous
import jax
import jax.numpy as jnp
from jax import lax
from jax.experimental import pallas as pl
from jax.experimental.pallas import tpu as pltpu

N_DEV = 32
M_PER = 64
HALF = M_PER // 2
D = 1024

R_HOPS = N_DEV // 2
L_HOPS = N_DEV // 2 - 1

CYCLE = [0, 3, 4, 7, 15, 12, 11, 8, 16, 19, 20, 23, 31, 28, 27, 24,
         25, 26, 29, 30, 22, 21, 18, 17, 9, 10, 13, 14, 6, 5, 2, 1]


def kernel(x, Win0, Wout0, Win1, Wout1, Win2, Wout2):
    xb = x.astype(jnp.bfloat16)
    wins = [w.astype(jnp.bfloat16) for w in (Win0, Win1, Win2)]
    wouts = [w.astype(jnp.bfloat16) for w in (Wout0, Wout1, Wout2)]
    cyc_tbl = jnp.asarray(CYCLE, dtype=jnp.int32).reshape(1, N_DEV)
    inv = [0] * N_DEV
    for k, v in enumerate(CYCLE):
        inv[v] = k
    inv_tbl = jnp.asarray(inv, dtype=jnp.int32).reshape(1, N_DEV)

    def body(cyc_ref, inv_ref, x_ref, win0_ref, win1_ref, win2_ref,
             wout0_ref, wout1_ref, wout2_ref, out_ref,
             xfull_ref, p_ref, sbuf_r_ref, sbuf_l_ref,
             rsr_ref, rsl_ref,
             ag_sr_sems, ag_sl_sems, rs_sr_sems, rs_sl_sems,
             ag_r_sems, ag_l_sems, rs_r_sems, rs_l_sems):
        me = lax.axis_index("i")

        def cyc(k):
            return cyc_ref[0, lax.rem(k + 2 * N_DEV, N_DEV)]

        c_me = inv_ref[0, me]
        left = cyc(c_me - 1)
        right = cyc(c_me + 1)

        barrier_sem = pltpu.get_barrier_semaphore()
        for nbr in (left, right):
            pl.semaphore_signal(
                barrier_sem, inc=1,
                device_id=(nbr,), device_id_type=pl.DeviceIdType.MESH,
            )
        pl.semaphore_wait(barrier_sem, 2)

        win_refs = (win0_ref, win1_ref, win2_ref)
        wout_refs = (wout0_ref, wout1_ref, wout2_ref)

        def ag_rdma(direction, half, h):
            if direction == 0:
                bj = cyc(c_me - h)
                tgt, ssem, rsem = right, ag_sr_sems, ag_r_sems
                n_steps = R_HOPS
            else:
                bj = cyc(c_me + h)
                tgt, ssem, rsem = left, ag_sl_sems, ag_l_sems
                n_steps = L_HOPS
            rows = pl.ds(bj * M_PER + half * HALF, HALF)
            return pltpu.make_async_remote_copy(
                src_ref=xfull_ref.at[rows],
                dst_ref=xfull_ref.at[rows],
                send_sem=ssem.at[half],
                recv_sem=rsem.at[half * n_steps + h],
                device_id=(tgt,),
                device_id_type=pl.DeviceIdType.MESH,
            )

        def rs_rdma(direction, half, s):
            if direction == 0:
                tgt, ssem, rsem = right, rs_sr_sems, rs_r_sems
                sbuf, slots, n_steps = sbuf_r_ref, rsr_ref, L_HOPS
            else:
                tgt, ssem, rsem = left, rs_sl_sems, rs_l_sems
                sbuf, slots, n_steps = sbuf_l_ref, rsl_ref, R_HOPS
            return pltpu.make_async_remote_copy(
                src_ref=sbuf.at[pl.ds(half * HALF, HALF)],
                dst_ref=slots.at[pl.ds(s * M_PER + half * HALF, HALF)],
                send_sem=ssem.at[half],
                recv_sem=rsem.at[half * n_steps + s],
                device_id=(tgt,),
                device_id_type=pl.DeviceIdType.MESH,
            )

        def compute_block(b, l):
            rows = pl.ds(b * M_PER, M_PER)
            xs = xfull_ref[rows, :]
            hh = jnp.dot(xs, win_refs[l][:, :],
                         preferred_element_type=jnp.float32)
            hh = jnp.maximum(hh, 0.0).astype(jnp.bfloat16)
            p_ref[rows, :] = jnp.dot(hh, wout_refs[l][:, :],
                                     preferred_element_type=jnp.float32)

        xfull_ref[pl.ds(me * M_PER, M_PER), :] = x_ref[:, :]

        for l in range(3):
            def ag_step(h, carry):
                for half in range(2):
                    @pl.when(h > 0)
                    def _():
                        ag_rdma(0, half, h - 1).wait()
                    ag_rdma(0, half, h).start()
                    @pl.when(h > 0)
                    def _():
                        ag_rdma(1, half, h - 1).wait()

                    @pl.when(h < L_HOPS)
                    def _():
                        ag_rdma(1, half, h).start()

                @pl.when(h == 0)
                def _():
                    compute_block(me, l)

                @pl.when(h > 0)
                def _():
                    compute_block(cyc(c_me - h), l)
                    compute_block(cyc(c_me + h), l)

                return carry

            lax.fori_loop(0, R_HOPS, ag_step, 0, unroll=False)
            for half in range(2):
                ag_rdma(0, half, R_HOPS - 1).wait()
            compute_block(cyc(c_me - R_HOPS), l)

            def rs_val(direction, half, s):
                if direction == 0:
                    bi = cyc(c_me + L_HOPS - s)
                    slots = rsr_ref
                else:
                    bi = cyc(c_me - R_HOPS + s)
                    slots = rsl_ref
                rows = pl.ds(bi * M_PER + half * HALF, HALF)
                own = p_ref[rows, :]
                sprev = jnp.maximum(s - 1, 0)
                prev = slots[pl.ds(sprev * M_PER + half * HALF, HALF), :]
                return jnp.where(
                    s == 0, own, own + prev.astype(jnp.float32)
                ).astype(jnp.bfloat16)

            def rs_step(s, carry):
                for half in range(2):
                    @pl.when(s > 0)
                    def _():
                        rs_rdma(1, half, s - 1).wait()
                    sbuf_l_ref[pl.ds(half * HALF, HALF), :] = rs_val(1, half, s)
                    rs_rdma(1, half, s).start()
                    @pl.when(s > 0)
                    def _():
                        rs_rdma(0, half, s - 1).wait()

                    @pl.when(s < L_HOPS)
                    def _():
                        sbuf_r_ref[pl.ds(half * HALF, HALF), :] = rs_val(
                            0, half, s)
                        rs_rdma(0, half, s).start()

                return carry

            lax.fori_loop(0, R_HOPS, rs_step, 0, unroll=False)
            for half in range(2):
                rs_rdma(1, half, R_HOPS - 1).wait()

            xnew = (p_ref[pl.ds(me * M_PER, M_PER), :]
                    + rsr_ref[pl.ds((L_HOPS - 1) * M_PER, M_PER),
                              :].astype(jnp.float32)
                    + rsl_ref[pl.ds((R_HOPS - 1) * M_PER, M_PER),
                              :].astype(jnp.float32))
            if l < 2:
                xfull_ref[pl.ds(me * M_PER, M_PER), :] = xnew.astype(
                    jnp.bfloat16)
            else:
                out_ref[:, :] = xnew

    return pl.pallas_call(
        body,
        out_shape=jax.ShapeDtypeStruct((M_PER, D), jnp.float32),
        in_specs=[pl.BlockSpec(memory_space=pltpu.SMEM)] * 2
        + [pl.BlockSpec(memory_space=pltpu.VMEM)] * 7,
        out_specs=pl.BlockSpec(memory_space=pltpu.VMEM),
        scratch_shapes=[
            pltpu.VMEM((N_DEV * M_PER, D), jnp.bfloat16),
            pltpu.VMEM((N_DEV * M_PER, D), jnp.float32),
            pltpu.VMEM((M_PER, D), jnp.bfloat16),
            pltpu.VMEM((M_PER, D), jnp.bfloat16),
            pltpu.VMEM((L_HOPS * M_PER, D), jnp.bfloat16),
            pltpu.VMEM((R_HOPS * M_PER, D), jnp.bfloat16),
            pltpu.SemaphoreType.DMA((2,)),
            pltpu.SemaphoreType.DMA((2,)),
            pltpu.SemaphoreType.DMA((2,)),
            pltpu.SemaphoreType.DMA((2,)),
            pltpu.SemaphoreType.DMA((2 * R_HOPS,)),
            pltpu.SemaphoreType.DMA((2 * L_HOPS,)),
            pltpu.SemaphoreType.DMA((2 * L_HOPS,)),
            pltpu.SemaphoreType.DMA((2 * R_HOPS,)),
        ],
        compiler_params=pltpu.CompilerParams(collective_id=0),
    )(cyc_tbl, inv_tbl, xb, wins[0], wins[1], wins[2],
      wouts[0], wouts[1], wouts[2])
